# Initial kernel scaffold; baseline (speedup 1.0000x reference)
import jax
import jax.numpy as jnp
from jax.experimental import pallas as pl


def kernel(X, X_original):
    raise NotImplementedError("write your pallas kernel here")


# speedup vs baseline: 2.4465x; 2.4465x over previous
import dataclasses
import functools

import jax
import jax.numpy as jnp
from jax.experimental import pallas as pl
from jax.experimental.pallas import tpu as pltpu
from jax.experimental.pallas import tpu_sc as plsc

_INTERIM = 256
_X_MIN = 0.0
_X_MAX = 1.4
_M = 128
_C_IN = 2 * _INTERIM + 4
_C_OUT = 2 * _M + 4
_L = 16
_CHUNK = 16


def _interp_block(x_vmem, q_vmem, o_vmem):
    @pl.loop(0, _CHUNK)
    def _row(r):
        row = jnp.full((_L,), r, dtype=jnp.int32)

        @pl.loop(0, _M, step=_L)
        def _grp(g):
            x = q_vmem[r, pl.ds(g, _L)]
            t = (x - _X_MIN) / (_X_MAX - _X_MIN) * float(_INTERIM - 1)
            t = jnp.minimum(jnp.maximum(t, 0.0), float(_INTERIM - 1))
            lo = jnp.minimum(t.astype(jnp.int32), _INTERIM - 2)
            frac = t - lo.astype(jnp.float32)
            p_lo = plsc.load_gather(x_vmem, [row, lo])
            p_hi = plsc.load_gather(x_vmem, [row, lo + 1])
            t_lo = plsc.load_gather(x_vmem, [row, lo + _INTERIM])
            t_hi = plsc.load_gather(x_vmem, [row, lo + (_INTERIM + 1)])
            o_vmem[r, pl.ds(g, _L)] = p_lo + frac * (p_hi - p_lo)
            o_vmem[r, pl.ds(g + _M, _L)] = t_lo + frac * (t_hi - t_lo)

    iota = jax.lax.iota(jnp.int32, _L)

    @pl.loop(0, _CHUNK, step=_L)
    def _pass(rb):
        rows = iota + rb
        for c in range(4):
            v = plsc.load_gather(
                x_vmem, [rows, jnp.full((_L,), 2 * _INTERIM + c, jnp.int32)])
            plsc.store_scatter(
                o_vmem, [rows, jnp.full((_L,), 2 * _M + c, jnp.int32)], v)


@functools.cache
def _build(batch):
    mesh = plsc.VectorSubcoreMesh(core_axis_name="c", subcore_axis_name="s")
    cp = pltpu.CompilerParams()
    if "needs_layout_passes" in pltpu.CompilerParams.__dataclass_fields__:
        cp = dataclasses.replace(cp, needs_layout_passes=False)

    @functools.partial(
        pl.kernel,
        out_type=jax.ShapeDtypeStruct((batch, _C_OUT), jnp.float32),
        mesh=mesh,
        compiler_params=cp,
    )
    def run(x_hbm, q_hbm, o_hbm):
        pltpu.emit_pipeline(
            _interp_block,
            grid=(batch // _CHUNK,),
            in_specs=[
                pl.BlockSpec((_CHUNK, _C_IN), lambda i: (i, 0)),
                pl.BlockSpec((_CHUNK, _M), lambda i: (i, 0)),
            ],
            out_specs=[pl.BlockSpec((_CHUNK, _C_OUT), lambda i: (i, 0))],
            core_axis_name=("c", "s"),
            dimension_semantics=(pltpu.PARALLEL,),
        )(x_hbm, q_hbm, o_hbm)

    return run


def kernel(X, X_original):
    return _build(X.shape[0])(X, X_original)


# speedup vs baseline: 2.5149x; 1.0280x over previous
import dataclasses
import functools

import jax
import jax.numpy as jnp
from jax.experimental import pallas as pl
from jax.experimental.pallas import tpu as pltpu
from jax.experimental.pallas import tpu_sc as plsc

_INTERIM = 256
_X_MIN = 0.0
_X_MAX = 1.4
_M = 128
_C_IN = 2 * _INTERIM + 4
_C_OUT = 2 * _M + 4
_L = 16
_CHUNK = 16


def _interp_block(x_vmem, q_vmem, o_vmem):
    @pl.loop(0, _CHUNK)
    def _row(r):
        row = jnp.full((_L,), r, dtype=jnp.int32)

        for g in range(0, _M, _L):
            x = q_vmem[r, pl.ds(g, _L)]
            t = (x - _X_MIN) / (_X_MAX - _X_MIN) * float(_INTERIM - 1)
            t = jnp.minimum(jnp.maximum(t, 0.0), float(_INTERIM - 1))
            lo = jnp.minimum(t.astype(jnp.int32), _INTERIM - 2)
            frac = t - lo.astype(jnp.float32)
            p_lo = plsc.load_gather(x_vmem, [row, lo])
            p_hi = plsc.load_gather(x_vmem, [row, lo + 1])
            t_lo = plsc.load_gather(x_vmem, [row, lo + _INTERIM])
            t_hi = plsc.load_gather(x_vmem, [row, lo + (_INTERIM + 1)])
            o_vmem[r, pl.ds(g, _L)] = p_lo + frac * (p_hi - p_lo)
            o_vmem[r, pl.ds(g + _M, _L)] = t_lo + frac * (t_hi - t_lo)

    iota = jax.lax.iota(jnp.int32, _L)

    @pl.loop(0, _CHUNK, step=_L)
    def _pass(rb):
        rows = iota + rb
        for c in range(4):
            v = plsc.load_gather(
                x_vmem, [rows, jnp.full((_L,), 2 * _INTERIM + c, jnp.int32)])
            plsc.store_scatter(
                o_vmem, [rows, jnp.full((_L,), 2 * _M + c, jnp.int32)], v)


@functools.cache
def _build(batch):
    mesh = plsc.VectorSubcoreMesh(core_axis_name="c", subcore_axis_name="s")
    cp = pltpu.CompilerParams()
    if "needs_layout_passes" in pltpu.CompilerParams.__dataclass_fields__:
        cp = dataclasses.replace(cp, needs_layout_passes=False)

    @functools.partial(
        pl.kernel,
        out_type=jax.ShapeDtypeStruct((batch, _C_OUT), jnp.float32),
        mesh=mesh,
        compiler_params=cp,
    )
    def run(x_hbm, q_hbm, o_hbm):
        pltpu.emit_pipeline(
            _interp_block,
            grid=(batch // _CHUNK,),
            in_specs=[
                pl.BlockSpec((_CHUNK, _C_IN), lambda i: (i, 0)),
                pl.BlockSpec((_CHUNK, _M), lambda i: (i, 0)),
            ],
            out_specs=[pl.BlockSpec((_CHUNK, _C_OUT), lambda i: (i, 0))],
            core_axis_name=("c", "s"),
            dimension_semantics=(pltpu.PARALLEL,),
        )(x_hbm, q_hbm, o_hbm)

    return run


def kernel(X, X_original):
    return _build(X.shape[0])(X, X_original)
